# baseline (speedup 1.0000x reference)
import functools

import jax
import jax.numpy as jnp
from jax import lax
from jax.experimental import pallas as pl
from jax.experimental.pallas import tpu as pltpu
from jax.experimental.pallas import tpu_sc as plsc

H = 128
AMPLIFY = 3.5
_F32 = jnp.float32

_NC, _NS, _LANES = 2, 16, 16
_NW = _NC * _NS
_KE = 128


def _gather_pre(at_tab, bn_tab, tgt_idx, nbr_idx):
    e = tgt_idx.shape[0]
    g = e // _NW
    mesh = plsc.VectorSubcoreMesh(core_axis_name="c", subcore_axis_name="s")

    @functools.partial(
        pl.kernel,
        mesh=mesh,
        out_type=jax.ShapeDtypeStruct((e, H), _F32),
        scratch_types=[
            pltpu.VMEM((_KE,), jnp.int32),
            pltpu.VMEM((_KE,), jnp.int32),
            pltpu.VMEM((_KE, H), _F32),
            pltpu.VMEM((_KE, H), _F32),
            pltpu.SemaphoreType.DMA,
            pltpu.SemaphoreType.DMA,
        ],
    )
    def k(at_hbm, bn_hbm, ti_hbm, ni_hbm, out_hbm, ia, ib, ra, rb, sa, sb):
        wid = lax.axis_index("s") * _NC + lax.axis_index("c")
        base = wid * g

        def body(ch, carry):
            eoff = base + ch * _KE
            pltpu.sync_copy(ti_hbm.at[pl.ds(eoff, _KE)], ia)
            pltpu.sync_copy(ni_hbm.at[pl.ds(eoff, _KE)], ib)
            ca = pltpu.async_copy(at_hbm.at[ia], ra, sa)
            cb = pltpu.async_copy(bn_hbm.at[ib], rb, sb)
            ca.wait()
            cb.wait()

            def add_row(r, c2):
                for j in range(H // _LANES):
                    sl = pl.ds(j * _LANES, _LANES)
                    ra[r, sl] = ra[r, sl] + rb[r, sl]
                return c2

            lax.fori_loop(0, _KE, add_row, 0, unroll=2)
            pltpu.sync_copy(ra, out_hbm.at[pl.ds(eoff, _KE)])
            return carry

        lax.fori_loop(0, g // _KE, body, 0)

    return k(at_tab, bn_tab, tgt_idx, nbr_idx)


def _mm_kernel(x_ref, w_ref, b_ref, o_ref, *, relu_in, relu_out):
    x = x_ref[...]
    if relu_in:
        x = jnp.maximum(x, 0.0)
    y = jnp.dot(x, w_ref[...], preferred_element_type=_F32) + b_ref[...]
    if relu_out:
        y = jnp.maximum(y, 0.0)
    o_ref[...] = y


def _mm(x, w, b, *, relu_in=False, relu_out=False, block_rows=2000):
    n, k = x.shape
    m = w.shape[1]
    npad = -n % block_rows
    if npad:
        x = jnp.pad(x, ((0, npad), (0, 0)))
    nt = n + npad
    kern = functools.partial(_mm_kernel, relu_in=relu_in, relu_out=relu_out)
    out = pl.pallas_call(
        kern,
        grid=(nt // block_rows,),
        in_specs=[
            pl.BlockSpec((block_rows, k), lambda i: (i, 0)),
            pl.BlockSpec((k, m), lambda i: (0, 0)),
            pl.BlockSpec((1, m), lambda i: (0, 0)),
        ],
        out_specs=pl.BlockSpec((block_rows, m), lambda i: (i, 0)),
        out_shape=jax.ShapeDtypeStruct((nt, m), _F32),
    )(x, w, b.reshape(1, m))
    return out[:n] if npad else out


def _node_kernel(deg_ref, s_ref, ss_ref, mx_ref, mn_ref, u1w_ref, u1b_ref,
                 o_ref):
    deg = deg_ref[...]
    degc = jnp.maximum(deg, 1.0)
    mean = s_ref[...] / degc
    var = jnp.maximum(ss_ref[...] / degc - mean * mean, 0.0)
    std = jnp.sqrt(var + 1e-5)
    has = deg > 0.0
    zero = jnp.zeros_like(mean)
    mx = jnp.where(has, mx_ref[...], zero)
    mn = jnp.where(has, mn_ref[...], zero)
    agg = jnp.concatenate([mx, mn, mean, std], axis=1)
    logd = jnp.log(deg + 1.0)
    amp = logd * (1.0 / AMPLIFY)
    att = jnp.where(has, AMPLIFY / jnp.maximum(logd, 1e-5), zero)
    amp4 = jnp.concatenate([amp] * 4, axis=1)
    att4 = jnp.concatenate([att] * 4, axis=1)
    scaled = jnp.concatenate([agg, agg * amp4, agg * att4], axis=1)
    h = jnp.dot(scaled, u1w_ref[...], preferred_element_type=_F32) + u1b_ref[...]
    o_ref[...] = jnp.maximum(h, 0.0)


def _pad_rows(x, nt):
    return x if x.shape[0] == nt else jnp.pad(x, ((0, nt - x.shape[0]), (0, 0)))


def _node(degb, s, ss, mx, mn, u1w, u1b, *, block_rows=1000):
    n = s.shape[0]
    block_rows = min(block_rows, -(-n // 8) * 8)
    nt = -(-n // block_rows) * block_rows
    degb, s, ss, mx, mn = (_pad_rows(a, nt) for a in (degb, s, ss, mx, mn))
    spec = pl.BlockSpec((block_rows, H), lambda i: (i, 0))
    out = pl.pallas_call(
        _node_kernel,
        grid=(nt // block_rows,),
        in_specs=[spec, spec, spec, spec, spec,
                  pl.BlockSpec((12 * H, H), lambda i: (0, 0)),
                  pl.BlockSpec((1, H), lambda i: (0, 0))],
        out_specs=spec,
        out_shape=jax.ShapeDtypeStruct((nt, H), _F32),
    )(degb, s, ss, mx, mn, u1w, u1b.reshape(1, H))
    return out[:n] if nt != n else out


def _head_kernel(x_ref, w1_ref, b1_ref, w2_ref, b2_ref, w3_ref, b3_ref,
                 o_ref):
    h = jnp.dot(x_ref[...], w1_ref[...], preferred_element_type=_F32) + b1_ref[...]
    h = jnp.maximum(h, 0.0)
    h = jnp.dot(h, w2_ref[...], preferred_element_type=_F32) + b2_ref[...]
    h = jnp.maximum(h, 0.0)
    z = jnp.dot(h, w3_ref[...], preferred_element_type=_F32) + b3_ref[...]
    o_ref[...] = 1.0 / (1.0 + jnp.exp(-z))


def _head(x, w1, b1, w2, b2, w3, b3, *, block_rows=2000):
    n = x.shape[0]
    block_rows = min(block_rows, -(-n // 8) * 8)
    nt = -(-n // block_rows) * block_rows
    x = _pad_rows(x, nt)
    spec = pl.BlockSpec((block_rows, H), lambda i: (i, 0))
    wspec = pl.BlockSpec((H, H), lambda i: (0, 0))
    bspec = pl.BlockSpec((1, H), lambda i: (0, 0))
    out = pl.pallas_call(
        _head_kernel,
        grid=(nt // block_rows,),
        in_specs=[spec, wspec, bspec, wspec, bspec, wspec, bspec],
        out_specs=spec,
        out_shape=jax.ShapeDtypeStruct((nt, H), _F32),
    )(x, w1, b1.reshape(1, H), w2, b2.reshape(1, H), w3, b3.reshape(1, H))
    return out[:n] if nt != n else out


def _pna_hidden(At, Bn, tgt_g, nbr_g, seg, n, degb, m2w, m2b, u1w, u1b):
    pre = _gather_pre(At, Bn, tgt_g, nbr_g)
    m = _mm(pre, m2w, m2b, relu_in=True, block_rows=2560)
    s = jax.ops.segment_sum(m, seg, num_segments=n + 1,
                            indices_are_sorted=True)[:n]
    ss = jax.ops.segment_sum(m * m, seg, num_segments=n + 1,
                             indices_are_sorted=True)[:n]
    mx = jax.ops.segment_max(m, seg, num_segments=n + 1,
                             indices_are_sorted=True)[:n]
    mn = -jax.ops.segment_max(-m, seg, num_segments=n + 1,
                              indices_are_sorted=True)[:n]
    return _node(degb, s, ss, mx, mn, u1w, u1b)


def kernel(demand, fac_init, adj, params):
    C = demand.shape[0]
    F = fac_init.shape[0]
    dst = adj[0]
    src = adj[1]

    E = dst.shape[0]
    perm_c = jnp.argsort(dst)
    dst_c = dst[perm_c]
    src_c = src[perm_c]
    perm_f = jnp.argsort(src)
    src_f = src[perm_f]
    dst_f = dst[perm_f]
    offs_c = jnp.searchsorted(dst_c, jnp.arange(C + 1, dtype=jnp.int32))
    deg_c = jnp.diff(offs_c).astype(_F32)
    offs_f = jnp.searchsorted(src_f, jnp.arange(F + 1, dtype=jnp.int32))
    deg_f = jnp.diff(offs_f).astype(_F32)
    degb_c = jnp.broadcast_to(deg_c[:, None], (C, H))
    degb_f = jnp.broadcast_to(deg_f[:, None], (F, H))

    ep = -(-E // (_NW * _KE)) * (_NW * _KE)

    def _padi(x, v):
        return jnp.pad(x, (0, ep - E), constant_values=v) if ep != E else x

    dstc_g, srcc_g, seg_c = _padi(dst_c, 0), _padi(src_c, 0), _padi(dst_c, C)
    srcf_g, dstf_g, seg_f = _padi(src_f, 0), _padi(dst_f, 0), _padi(src_f, F)

    p = params
    wce, bce = p["cus_emv"]["W"], p["cus_emv"]["b"]
    wfe, bfe = p["fac_emv"]["W"], p["fac_emv"]["b"]
    cp, fp = p["cus_pna"], p["fac_pna"]
    wt_c, wn_c, b1_c = cp["M1"]["W"][:H], cp["M1"]["W"][H:], cp["M1"]["b"]
    wt_f, wn_f, b1_f = fp["M1"]["W"][:H], fp["M1"]["W"][H:], fp["M1"]["b"]
    m2w_c, m2b_c = cp["M2"]["W"], cp["M2"]["b"]
    m2w_f, m2b_f = fp["M2"]["W"], fp["M2"]["b"]
    u1w_c, u1b_c = cp["U1"]["W"], cp["U1"]["b"]
    u1w_f, u1b_f = fp["U1"]["W"], fp["U1"]["b"]
    u2w_c, u2b_c = cp["U2"]["W"], cp["U2"]["b"]
    u2w_f, u2b_f = fp["U2"]["W"], fp["U2"]["b"]

    w_atc, b_atc = u2w_c @ wt_c, u2b_c @ wt_c + b1_c
    w_bnc, b_bnc = u2w_c @ wn_f, u2b_c @ wn_f
    w_atf, b_atf = u2w_f @ wt_f, u2b_f @ wt_f + b1_f
    w_bnf, b_bnf = u2w_f @ wn_c, u2b_f @ wn_c

    at_c = _mm(demand, wce @ wt_c, bce @ wt_c + b1_c)
    bn_c = _mm(demand, wce @ wn_f, bce @ wn_f)
    at_f = _mm(fac_init, wfe @ wt_f, bfe @ wt_f + b1_f)
    bn_f = _mm(fac_init, wfe @ wn_c, bfe @ wn_c)

    h_c = _pna_hidden(at_c, bn_f, dstc_g, srcc_g, seg_c, C, degb_c,
                      m2w_c, m2b_c, u1w_c, u1b_c)
    h_f = _pna_hidden(at_f, bn_c, srcf_g, dstf_g, seg_f, F, degb_f,
                      m2w_f, m2b_f, u1w_f, u1b_f)
    at_c = _mm(h_c, w_atc, b_atc)
    bn_c = _mm(h_c, w_bnc, b_bnc)
    at_f = _mm(h_f, w_atf, b_atf)
    bn_f = _mm(h_f, w_bnf, b_bnf)

    h_c = _pna_hidden(at_c, bn_f, dstc_g, srcc_g, seg_c, C, degb_c,
                      m2w_c, m2b_c, u1w_c, u1b_c)
    h_f = _pna_hidden(at_f, bn_c, srcf_g, dstf_g, seg_f, F, degb_f,
                      m2w_f, m2b_f, u1w_f, u1b_f)
    bn_c = _mm(h_c, w_bnc, b_bnc)
    at_f = _mm(h_f, w_atf, b_atf)

    h_f = _pna_hidden(at_f, bn_c, srcf_g, dstf_g, seg_f, F, degb_f,
                      m2w_f, m2b_f, u1w_f, u1b_f)
    f3 = _mm(h_f, u2w_f, u2b_f)

    f1w = jnp.zeros((H, H), _F32).at[:, :12].set(p["f1"]["W"])
    f1b = jnp.zeros((H,), _F32).at[:12].set(p["f1"]["b"])
    f2w = jnp.zeros((H, H), _F32).at[:12, :12].set(p["f2"]["W"])
    f2b = jnp.zeros((H,), _F32).at[:12].set(p["f2"]["b"])
    f3w = jnp.zeros((H, H), _F32).at[:12, :1].set(p["f3"]["W"])
    f3b = jnp.zeros((H,), _F32).at[:1].set(p["f3"]["b"])
    out = _head(f3, f1w, f1b, f2w, f2b, f3w, f3b)
    return out[:, :1]


# speedup vs baseline: 2.1821x; 2.1821x over previous
import functools

import jax
import jax.numpy as jnp
from jax import lax
from jax.experimental import pallas as pl
from jax.experimental.pallas import tpu as pltpu
from jax.experimental.pallas import tpu_sc as plsc

H = 128
AMPLIFY = 3.5
_F32 = jnp.float32

_NC, _NS, _LANES = 2, 16, 16
_NW = _NC * _NS
_KE = 128


def _gather_pre(at_tab, bn_tab, tgt_idx, nbr_idx):
    e = tgt_idx.shape[0]
    g = e // _NW
    mesh = plsc.VectorSubcoreMesh(core_axis_name="c", subcore_axis_name="s")

    @functools.partial(
        pl.kernel,
        mesh=mesh,
        out_type=jax.ShapeDtypeStruct((e, H), _F32),
        scratch_types=[
            pltpu.VMEM((_KE,), jnp.int32),
            pltpu.VMEM((_KE,), jnp.int32),
            pltpu.VMEM((_KE, H), _F32),
            pltpu.VMEM((_KE, H), _F32),
            pltpu.SemaphoreType.DMA,
            pltpu.SemaphoreType.DMA,
        ],
    )
    def k(at_hbm, bn_hbm, ti_hbm, ni_hbm, out_hbm, ia, ib, ra, rb, sa, sb):
        wid = lax.axis_index("s") * _NC + lax.axis_index("c")
        base = wid * g

        def body(ch, carry):
            eoff = base + ch * _KE
            pltpu.sync_copy(ti_hbm.at[pl.ds(eoff, _KE)], ia)
            pltpu.sync_copy(ni_hbm.at[pl.ds(eoff, _KE)], ib)
            ca = pltpu.async_copy(at_hbm.at[ia], ra, sa)
            cb = pltpu.async_copy(bn_hbm.at[ib], rb, sb)
            ca.wait()
            cb.wait()

            def add_row(r, c2):
                for j in range(H // _LANES):
                    sl = pl.ds(j * _LANES, _LANES)
                    ra[r, sl] = ra[r, sl] + rb[r, sl]
                return c2

            lax.fori_loop(0, _KE, add_row, 0, unroll=2)
            pltpu.sync_copy(ra, out_hbm.at[pl.ds(eoff, _KE)])
            return carry

        lax.fori_loop(0, g // _KE, body, 0)

    return k(at_tab, bn_tab, tgt_idx, nbr_idx)


_NPP = 160
_CH = 128


def _seg_reduce(m, seg, offs):
    npad = 2 * _NW * _NPP
    row_w = 4 * H
    mesh = plsc.VectorSubcoreMesh(core_axis_name="c", subcore_axis_name="s")

    @functools.partial(
        pl.kernel,
        mesh=mesh,
        out_type=jax.ShapeDtypeStruct((npad * row_w,), _F32),
        scratch_types=[
            pltpu.VMEM((_NPP * row_w,), _F32),
            pltpu.VMEM((_CH * H,), _F32),
            pltpu.VMEM((_CH + 16,), jnp.int32),
            pltpu.VMEM((16,), jnp.int32),
            pltpu.VMEM((16,), jnp.int32),
        ],
    )
    def k(m_hbm, seg_hbm, offs_hbm, out_hbm, stg, mbuf, sbuf, ob0, ob1):
        wid = lax.axis_index("s") * _NC + lax.axis_index("c")
        zero = jnp.zeros((_LANES,), _F32)
        ident = ((zero,) * 16
                 + (jnp.full((_LANES,), -3e38, _F32),) * 8
                 + (jnp.full((_LANES,), 3e38, _F32),) * 8)

        def flush(cur, accs, v0):
            base = (cur - v0) * row_w
            for j in range(8):
                for t in range(4):
                    stg[pl.ds(base + t * H + j * _LANES, _LANES)] = \
                        accs[8 * t + j]

        for p in range(2):
            v0 = pl.multiple_of((wid * 2 + p) * _NPP, 32)

            def zrow(r, c):
                for j in range(row_w // _LANES):
                    stg[pl.ds(r * row_w + j * _LANES, _LANES)] = zero
                return c

            lax.fori_loop(0, _NPP, zrow, 0)
            pltpu.sync_copy(offs_hbm.at[pl.ds(v0, 16)], ob0)
            pltpu.sync_copy(offs_hbm.at[pl.ds(v0 + _NPP, 16)], ob1)
            e0 = ob0[pl.ds(0, 16)][0]
            e1 = ob1[pl.ds(0, 16)][0]
            a0 = pl.multiple_of((e0 // 16) * 16, 16)
            nch = (e1 - a0 + (_CH - 1)) // _CH

            def chunk_body(kk, carry):
                cs = pl.multiple_of(a0 + kk * _CH, 16)
                r_lo = jnp.maximum(0, e0 - cs)
                r_hi = jnp.minimum(_CH, e1 - cs)
                pltpu.sync_copy(m_hbm.at[pl.ds(cs * H, _CH * H)], mbuf)
                pltpu.sync_copy(seg_hbm.at[pl.ds(cs, _CH + 16)], sbuf)

                def edge_body(r, ec):
                    sv = sbuf[pl.ds(r, 16)][0]

                    def do_flush(ops):
                        flush(ops[0], ops[1:], v0)
                        return (sv,) + ident

                    def no_flush(ops):
                        return (sv,) + ops[1:]

                    st = lax.cond(
                        jnp.logical_and(sv != ec[0], ec[0] >= 0),
                        do_flush, no_flush, ec)
                    new = list(st)
                    for j in range(8):
                        v = mbuf[pl.ds(r * H + j * _LANES, _LANES)]
                        new[1 + j] = st[1 + j] + v
                        new[9 + j] = st[9 + j] + v * v
                        new[17 + j] = jnp.maximum(st[17 + j], v)
                        new[25 + j] = jnp.minimum(st[25 + j], v)
                    return tuple(new)

                return lax.fori_loop(r_lo, r_hi, edge_body, carry)

            fin = lax.fori_loop(0, nch, chunk_body, (jnp.int32(-1),) + ident)

            def last_flush(ops):
                flush(ops[0], ops[1:], v0)
                return ops

            lax.cond(fin[0] >= 0, last_flush, lambda o: o, fin)
            pltpu.sync_copy(stg, out_hbm.at[pl.ds(v0 * row_w, _NPP * row_w)])

    return k(m.reshape(-1), seg, offs).reshape(npad, row_w)


def _mm_kernel(x_ref, w_ref, b_ref, o_ref, *, relu_in, relu_out):
    x = x_ref[...]
    if relu_in:
        x = jnp.maximum(x, 0.0)
    y = jnp.dot(x, w_ref[...], preferred_element_type=_F32) + b_ref[...]
    if relu_out:
        y = jnp.maximum(y, 0.0)
    o_ref[...] = y


def _mm(x, w, b, *, relu_in=False, relu_out=False, block_rows=2000):
    n, k = x.shape
    m = w.shape[1]
    npad = -n % block_rows
    if npad:
        x = jnp.pad(x, ((0, npad), (0, 0)))
    nt = n + npad
    kern = functools.partial(_mm_kernel, relu_in=relu_in, relu_out=relu_out)
    out = pl.pallas_call(
        kern,
        grid=(nt // block_rows,),
        in_specs=[
            pl.BlockSpec((block_rows, k), lambda i: (i, 0)),
            pl.BlockSpec((k, m), lambda i: (0, 0)),
            pl.BlockSpec((1, m), lambda i: (0, 0)),
        ],
        out_specs=pl.BlockSpec((block_rows, m), lambda i: (i, 0)),
        out_shape=jax.ShapeDtypeStruct((nt, m), _F32),
    )(x, w, b.reshape(1, m))
    return out[:n] if npad else out


def _node_kernel(deg_ref, tab_ref, u1w_ref, u1b_ref, o_ref):
    deg = deg_ref[...]
    tab = tab_ref[...]
    s = tab[:, :H]
    ss = tab[:, H:2 * H]
    degc = jnp.maximum(deg, 1.0)
    mean = s / degc
    var = jnp.maximum(ss / degc - mean * mean, 0.0)
    std = jnp.sqrt(var + 1e-5)
    has = deg > 0.0
    zero = jnp.zeros_like(mean)
    mx = jnp.where(has, tab[:, 2 * H:3 * H], zero)
    mn = jnp.where(has, tab[:, 3 * H:], zero)
    agg = jnp.concatenate([mx, mn, mean, std], axis=1)
    logd = jnp.log(deg + 1.0)
    amp = logd * (1.0 / AMPLIFY)
    att = jnp.where(has, AMPLIFY / jnp.maximum(logd, 1e-5), zero)
    amp4 = jnp.concatenate([amp] * 4, axis=1)
    att4 = jnp.concatenate([att] * 4, axis=1)
    scaled = jnp.concatenate([agg, agg * amp4, agg * att4], axis=1)
    h = jnp.dot(scaled, u1w_ref[...], preferred_element_type=_F32) + u1b_ref[...]
    o_ref[...] = jnp.maximum(h, 0.0)


def _pad_rows(x, nt):
    return x if x.shape[0] == nt else jnp.pad(x, ((0, nt - x.shape[0]), (0, 0)))


def _node(degb, tab, u1w, u1b, *, block_rows=1024):
    n = tab.shape[0]
    block_rows = min(block_rows, -(-n // 8) * 8)
    nt = -(-n // block_rows) * block_rows
    degb = _pad_rows(degb, nt)
    tab = _pad_rows(tab, nt)
    spec = pl.BlockSpec((block_rows, H), lambda i: (i, 0))
    out = pl.pallas_call(
        _node_kernel,
        grid=(nt // block_rows,),
        in_specs=[spec,
                  pl.BlockSpec((block_rows, 4 * H), lambda i: (i, 0)),
                  pl.BlockSpec((12 * H, H), lambda i: (0, 0)),
                  pl.BlockSpec((1, H), lambda i: (0, 0))],
        out_specs=spec,
        out_shape=jax.ShapeDtypeStruct((nt, H), _F32),
    )(degb, tab, u1w, u1b.reshape(1, H))
    return out[:n] if nt != n else out


def _head_kernel(x_ref, w1_ref, b1_ref, w2_ref, b2_ref, w3_ref, b3_ref,
                 o_ref):
    h = jnp.dot(x_ref[...], w1_ref[...], preferred_element_type=_F32) + b1_ref[...]
    h = jnp.maximum(h, 0.0)
    h = jnp.dot(h, w2_ref[...], preferred_element_type=_F32) + b2_ref[...]
    h = jnp.maximum(h, 0.0)
    z = jnp.dot(h, w3_ref[...], preferred_element_type=_F32) + b3_ref[...]
    o_ref[...] = 1.0 / (1.0 + jnp.exp(-z))


def _head(x, w1, b1, w2, b2, w3, b3, *, block_rows=2000):
    n = x.shape[0]
    block_rows = min(block_rows, -(-n // 8) * 8)
    nt = -(-n // block_rows) * block_rows
    x = _pad_rows(x, nt)
    spec = pl.BlockSpec((block_rows, H), lambda i: (i, 0))
    wspec = pl.BlockSpec((H, H), lambda i: (0, 0))
    bspec = pl.BlockSpec((1, H), lambda i: (0, 0))
    out = pl.pallas_call(
        _head_kernel,
        grid=(nt // block_rows,),
        in_specs=[spec, wspec, bspec, wspec, bspec, wspec, bspec],
        out_specs=spec,
        out_shape=jax.ShapeDtypeStruct((nt, H), _F32),
    )(x, w1, b1.reshape(1, H), w2, b2.reshape(1, H), w3, b3.reshape(1, H))
    return out[:n] if nt != n else out


def _pna_hidden(At, Bn, tgt_g, nbr_g, seg, offs, degb, m2w, m2b, u1w, u1b):
    pre = _gather_pre(At, Bn, tgt_g, nbr_g)
    m = _mm(pre, m2w, m2b, relu_in=True, block_rows=2560)
    tab = _seg_reduce(m, seg, offs)
    return _node(degb, tab, u1w, u1b)


def kernel(demand, fac_init, adj, params):
    C = demand.shape[0]
    F = fac_init.shape[0]
    dst = adj[0]
    src = adj[1]

    E = dst.shape[0]
    perm_c = jnp.argsort(dst)
    dst_c = dst[perm_c]
    src_c = src[perm_c]
    perm_f = jnp.argsort(src)
    src_f = src[perm_f]
    dst_f = dst[perm_f]
    offs_c = jnp.searchsorted(dst_c, jnp.arange(C + 1, dtype=jnp.int32))
    deg_c = jnp.diff(offs_c).astype(_F32)
    offs_f = jnp.searchsorted(src_f, jnp.arange(F + 1, dtype=jnp.int32))
    deg_f = jnp.diff(offs_f).astype(_F32)
    degb_c = jnp.broadcast_to(deg_c[:, None], (C, H))
    degb_f = jnp.broadcast_to(deg_f[:, None], (F, H))

    ep = -(-(E + 160) // (_NW * _KE)) * (_NW * _KE)

    def _padi(x, v):
        return jnp.pad(x, (0, ep - E), constant_values=v) if ep != E else x

    dstc_g, srcc_g, seg_c = _padi(dst_c, 0), _padi(src_c, 0), _padi(dst_c, C)
    srcf_g, dstf_g, seg_f = _padi(src_f, 0), _padi(dst_f, 0), _padi(src_f, F)

    npad = 2 * _NW * _NPP
    osz = npad + 16

    def _pado(offs, n):
        return jnp.concatenate(
            [offs, jnp.full((osz - n - 1,), E, jnp.int32)]).astype(jnp.int32)

    offs_ec = _pado(offs_c, C)
    offs_ef = _pado(offs_f, F)
    degb_c = _pad_rows(degb_c, npad)
    degb_f = _pad_rows(degb_f, npad)

    p = params
    wce, bce = p["cus_emv"]["W"], p["cus_emv"]["b"]
    wfe, bfe = p["fac_emv"]["W"], p["fac_emv"]["b"]
    cp, fp = p["cus_pna"], p["fac_pna"]
    wt_c, wn_c, b1_c = cp["M1"]["W"][:H], cp["M1"]["W"][H:], cp["M1"]["b"]
    wt_f, wn_f, b1_f = fp["M1"]["W"][:H], fp["M1"]["W"][H:], fp["M1"]["b"]
    m2w_c, m2b_c = cp["M2"]["W"], cp["M2"]["b"]
    m2w_f, m2b_f = fp["M2"]["W"], fp["M2"]["b"]
    u1w_c, u1b_c = cp["U1"]["W"], cp["U1"]["b"]
    u1w_f, u1b_f = fp["U1"]["W"], fp["U1"]["b"]
    u2w_c, u2b_c = cp["U2"]["W"], cp["U2"]["b"]
    u2w_f, u2b_f = fp["U2"]["W"], fp["U2"]["b"]

    w_atc, b_atc = u2w_c @ wt_c, u2b_c @ wt_c + b1_c
    w_bnc, b_bnc = u2w_c @ wn_f, u2b_c @ wn_f
    w_atf, b_atf = u2w_f @ wt_f, u2b_f @ wt_f + b1_f
    w_bnf, b_bnf = u2w_f @ wn_c, u2b_f @ wn_c

    at_c = _mm(demand, wce @ wt_c, bce @ wt_c + b1_c)
    bn_c = _mm(demand, wce @ wn_f, bce @ wn_f)
    at_f = _mm(fac_init, wfe @ wt_f, bfe @ wt_f + b1_f)
    bn_f = _mm(fac_init, wfe @ wn_c, bfe @ wn_c)

    h_c = _pna_hidden(at_c, bn_f, dstc_g, srcc_g, seg_c, offs_ec, degb_c,
                      m2w_c, m2b_c, u1w_c, u1b_c)
    h_f = _pna_hidden(at_f, bn_c, srcf_g, dstf_g, seg_f, offs_ef, degb_f,
                      m2w_f, m2b_f, u1w_f, u1b_f)
    at_c = _mm(h_c, w_atc, b_atc)
    bn_c = _mm(h_c, w_bnc, b_bnc)
    at_f = _mm(h_f, w_atf, b_atf)
    bn_f = _mm(h_f, w_bnf, b_bnf)

    h_c = _pna_hidden(at_c, bn_f, dstc_g, srcc_g, seg_c, offs_ec, degb_c,
                      m2w_c, m2b_c, u1w_c, u1b_c)
    h_f = _pna_hidden(at_f, bn_c, srcf_g, dstf_g, seg_f, offs_ef, degb_f,
                      m2w_f, m2b_f, u1w_f, u1b_f)
    bn_c = _mm(h_c, w_bnc, b_bnc)
    at_f = _mm(h_f, w_atf, b_atf)

    h_f = _pna_hidden(at_f, bn_c, srcf_g, dstf_g, seg_f, offs_ef, degb_f,
                      m2w_f, m2b_f, u1w_f, u1b_f)
    f3 = _mm(h_f, u2w_f, u2b_f)

    f1w = jnp.zeros((H, H), _F32).at[:, :12].set(p["f1"]["W"])
    f1b = jnp.zeros((H,), _F32).at[:12].set(p["f1"]["b"])
    f2w = jnp.zeros((H, H), _F32).at[:12, :12].set(p["f2"]["W"])
    f2b = jnp.zeros((H,), _F32).at[:12].set(p["f2"]["b"])
    f3w = jnp.zeros((H, H), _F32).at[:12, :1].set(p["f3"]["W"])
    f3b = jnp.zeros((H,), _F32).at[:1].set(p["f3"]["b"])
    out = _head(f3, f1w, f1b, f2w, f2b, f3w, f3b)
    return out[:F, :1]


# speedup vs baseline: 2.6697x; 1.2234x over previous
import functools

import jax
import jax.numpy as jnp
from jax import lax
from jax.experimental import pallas as pl
from jax.experimental.pallas import tpu as pltpu
from jax.experimental.pallas import tpu_sc as plsc

H = 128
AMPLIFY = 3.5
_F32 = jnp.float32

_NC, _NS, _LANES = 2, 16, 16
_NW = _NC * _NS
_KE = 128


def _gather_pre(at_tab, bn_tab, tgt_idx, nbr_idx):
    e = tgt_idx.shape[0]
    g = e // _NW
    nch = g // _KE
    mesh = plsc.VectorSubcoreMesh(core_axis_name="c", subcore_axis_name="s")

    @functools.partial(
        pl.kernel,
        mesh=mesh,
        out_type=jax.ShapeDtypeStruct((e, H), _F32),
        scratch_types=[
            pltpu.VMEM((g,), jnp.int32),
            pltpu.VMEM((g,), jnp.int32),
            pltpu.VMEM((_KE, H), _F32),
            pltpu.VMEM((_KE, H), _F32),
            pltpu.VMEM((_KE, H), _F32),
            pltpu.VMEM((_KE, H), _F32),
            pltpu.VMEM((_KE, H), _F32),
            pltpu.VMEM((_KE, H), _F32),
            pltpu.SemaphoreType.DMA,
            pltpu.SemaphoreType.DMA,
            pltpu.SemaphoreType.DMA,
            pltpu.SemaphoreType.DMA,
            pltpu.SemaphoreType.DMA,
            pltpu.SemaphoreType.DMA,
        ],
    )
    def k(at_hbm, bn_hbm, ti_hbm, ni_hbm, out_hbm,
          ia, ib, ra0, ra1, rb0, rb1, ro0, ro1,
          sa0, sa1, sb0, sb1, so0, so1):
        wid = lax.axis_index("s") * _NC + lax.axis_index("c")
        base = wid * g
        ras, rbs, ros = (ra0, ra1), (rb0, rb1), (ro0, ro1)
        sas, sbs, sos = (sa0, sa1), (sb0, sb1), (so0, so1)
        pltpu.sync_copy(ti_hbm.at[pl.ds(base, g)], ia)
        pltpu.sync_copy(ni_hbm.at[pl.ds(base, g)], ib)

        def issue(ch, b):
            c = jnp.minimum(ch, nch - 1)
            sl = pl.ds(c * _KE, _KE)
            pltpu.async_copy(at_hbm.at[ia.at[sl]], ras[b], sas[b])
            pltpu.async_copy(bn_hbm.at[ib.at[sl]], rbs[b], sbs[b])

        def wait_g(b):
            pltpu.make_async_copy(at_hbm.at[pl.ds(0, _KE)], ras[b],
                                  sas[b]).wait()
            pltpu.make_async_copy(bn_hbm.at[pl.ds(0, _KE)], rbs[b],
                                  sbs[b]).wait()

        def compute(b):
            ra, rb, ro = ras[b], rbs[b], ros[b]

            def add_row(r, c2):
                for j in range(H // _LANES):
                    sl = pl.ds(j * _LANES, _LANES)
                    ro[r, sl] = ra[r, sl] + rb[r, sl]
                return c2

            lax.fori_loop(0, _KE, add_row, 0, unroll=2)

        def out_dma(ch, b):
            eoff = base + ch * _KE
            return pltpu.async_copy(ros[b], out_hbm.at[pl.ds(eoff, _KE)],
                                    sos[b])

        def wait_o(b):
            pltpu.make_async_copy(ros[b], out_hbm.at[pl.ds(0, _KE)],
                                  sos[b]).wait()

        if nch < 4:
            def sbody(ch, carry):
                issue(ch, 0)
                wait_g(0)
                compute(0)
                out_dma(ch, 0)
                wait_o(0)
                return carry

            lax.fori_loop(0, nch, sbody, 0)
            return

        issue(0, 0)
        issue(1, 1)
        wait_g(0)
        compute(0)
        out_dma(0, 0)
        issue(2, 0)
        wait_g(1)
        compute(1)
        out_dma(1, 1)
        issue(3, 1)

        def body(p, carry):
            for b in range(2):
                ch = 2 * p + b
                wait_g(b)
                wait_o(b)
                compute(b)
                out_dma(ch, b)
                issue(ch + 2, b)
            return carry

        lax.fori_loop(1, nch // 2, body, 0)
        if nch % 2:
            wait_g(0)
            wait_o(0)
            compute(0)
            out_dma(nch - 1, 0)
            wait_o(0)
            wait_o(1)
            wait_g(1)
        else:
            wait_o(0)
            wait_o(1)
            wait_g(0)
            wait_g(1)

    return k(at_tab, bn_tab, tgt_idx, nbr_idx)


_NPP = 160
_CH = 128


def _seg_reduce(m, seg, offs):
    npad = 2 * _NW * _NPP
    row_w = 4 * H
    mesh = plsc.VectorSubcoreMesh(core_axis_name="c", subcore_axis_name="s")

    @functools.partial(
        pl.kernel,
        mesh=mesh,
        out_type=jax.ShapeDtypeStruct((npad * row_w,), _F32),
        scratch_types=[
            pltpu.VMEM((_NPP * row_w,), _F32),
            pltpu.VMEM((_CH * H,), _F32),
            pltpu.VMEM((_CH * H,), _F32),
            pltpu.VMEM((_CH + 16,), jnp.int32),
            pltpu.VMEM((_CH + 16,), jnp.int32),
            pltpu.VMEM((16,), jnp.int32),
            pltpu.VMEM((16,), jnp.int32),
            pltpu.SemaphoreType.DMA,
            pltpu.SemaphoreType.DMA,
            pltpu.SemaphoreType.DMA,
            pltpu.SemaphoreType.DMA,
        ],
    )
    def k(m_hbm, seg_hbm, offs_hbm, out_hbm, stg, mb0, mb1, sb0, sb1,
          ob0, ob1, sm0, sm1, sg0, sg1):
        mbufs, sbufs = (mb0, mb1), (sb0, sb1)
        sms, sgs = (sm0, sm1), (sg0, sg1)
        wid = lax.axis_index("s") * _NC + lax.axis_index("c")
        zero = jnp.zeros((_LANES,), _F32)
        ident = ((zero,) * 16
                 + (jnp.full((_LANES,), -3e38, _F32),) * 8
                 + (jnp.full((_LANES,), 3e38, _F32),) * 8)

        def flush(cur, accs, v0):
            base = (cur - v0) * row_w
            for j in range(8):
                for t in range(4):
                    stg[pl.ds(base + t * H + j * _LANES, _LANES)] = \
                        accs[8 * t + j]

        for p in range(2):
            v0 = pl.multiple_of((wid * 2 + p) * _NPP, 32)

            def zrow(r, c):
                for j in range(row_w // _LANES):
                    stg[pl.ds(r * row_w + j * _LANES, _LANES)] = zero
                return c

            lax.fori_loop(0, _NPP, zrow, 0)
            pltpu.sync_copy(offs_hbm.at[pl.ds(v0, 16)], ob0)
            pltpu.sync_copy(offs_hbm.at[pl.ds(v0 + _NPP, 16)], ob1)
            e0 = ob0[pl.ds(0, 16)][0]
            e1 = ob1[pl.ds(0, 16)][0]
            a0 = pl.multiple_of((e0 // 16) * 16, 16)
            nch = (e1 - a0 + (_CH - 1)) // _CH

            def issue_r(g, b):
                c = jnp.minimum(g, jnp.maximum(nch - 1, 0))
                cs = pl.multiple_of((a0 + c * _CH) // 16 * 16, 16)
                pltpu.async_copy(m_hbm.at[pl.ds(cs * H, _CH * H)],
                                 mbufs[b], sms[b])
                pltpu.async_copy(seg_hbm.at[pl.ds(cs, _CH + 16)],
                                 sbufs[b], sgs[b])

            def wait_r(b):
                pltpu.make_async_copy(m_hbm.at[pl.ds(0, _CH * H)],
                                      mbufs[b], sms[b]).wait()
                pltpu.make_async_copy(seg_hbm.at[pl.ds(0, _CH + 16)],
                                      sbufs[b], sgs[b]).wait()

            def process(g, b, carry):
                mbuf, sbuf = mbufs[b], sbufs[b]
                cs = a0 + g * _CH
                r_lo = jnp.maximum(0, e0 - cs)
                r_hi = jnp.minimum(_CH, e1 - cs)

                def edge_body(r, ec):
                    sv = sbuf[pl.ds(r, 16)][0]

                    def do_flush(ops):
                        flush(ops[0], ops[1:], v0)
                        return (sv,) + ident

                    def no_flush(ops):
                        return (sv,) + ops[1:]

                    st = lax.cond(
                        jnp.logical_and(sv != ec[0], ec[0] >= 0),
                        do_flush, no_flush, ec)
                    new = list(st)
                    for j in range(8):
                        v = mbuf[pl.ds(r * H + j * _LANES, _LANES)]
                        new[1 + j] = st[1 + j] + v
                        new[9 + j] = st[9 + j] + v * v
                        new[17 + j] = jnp.maximum(st[17 + j], v)
                        new[25 + j] = jnp.minimum(st[25 + j], v)
                    return tuple(new)

                return lax.fori_loop(r_lo, r_hi, edge_body, carry)

            issue_r(0, 0)

            def pair_body(p, carry):
                for b in range(2):
                    g = 2 * p + b
                    issue_r(g + 1, 1 - b)
                    wait_r(b)
                    carry = process(g, b, carry)
                return carry

            fin = lax.fori_loop(0, (nch + 1) // 2, pair_body,
                                (jnp.int32(-1),) + ident)
            wait_r(0)

            def last_flush(ops):
                flush(ops[0], ops[1:], v0)
                return ops

            lax.cond(fin[0] >= 0, last_flush, lambda o: o, fin)
            pltpu.sync_copy(stg, out_hbm.at[pl.ds(v0 * row_w, _NPP * row_w)])

    return k(m.reshape(-1), seg, offs).reshape(npad, row_w)


def _mm_kernel(x_ref, w_ref, b_ref, o_ref, *, relu_in, relu_out):
    x = x_ref[...]
    if relu_in:
        x = jnp.maximum(x, 0.0)
    y = jnp.dot(x, w_ref[...], preferred_element_type=_F32) + b_ref[...]
    if relu_out:
        y = jnp.maximum(y, 0.0)
    o_ref[...] = y


def _mm(x, w, b, *, relu_in=False, relu_out=False, block_rows=2000):
    n, k = x.shape
    m = w.shape[1]
    npad = -n % block_rows
    if npad:
        x = jnp.pad(x, ((0, npad), (0, 0)))
    nt = n + npad
    kern = functools.partial(_mm_kernel, relu_in=relu_in, relu_out=relu_out)
    out = pl.pallas_call(
        kern,
        grid=(nt // block_rows,),
        in_specs=[
            pl.BlockSpec((block_rows, k), lambda i: (i, 0)),
            pl.BlockSpec((k, m), lambda i: (0, 0)),
            pl.BlockSpec((1, m), lambda i: (0, 0)),
        ],
        out_specs=pl.BlockSpec((block_rows, m), lambda i: (i, 0)),
        out_shape=jax.ShapeDtypeStruct((nt, m), _F32),
    )(x, w, b.reshape(1, m))
    return out[:n] if npad else out


def _node_kernel(deg_ref, tab_ref, u1w_ref, u1b_ref, o_ref):
    deg = deg_ref[...]
    tab = tab_ref[...]
    s = tab[:, :H]
    ss = tab[:, H:2 * H]
    degc = jnp.maximum(deg, 1.0)
    mean = s / degc
    var = jnp.maximum(ss / degc - mean * mean, 0.0)
    std = jnp.sqrt(var + 1e-5)
    has = deg > 0.0
    zero = jnp.zeros_like(mean)
    mx = jnp.where(has, tab[:, 2 * H:3 * H], zero)
    mn = jnp.where(has, tab[:, 3 * H:], zero)
    agg = jnp.concatenate([mx, mn, mean, std], axis=1)
    logd = jnp.log(deg + 1.0)
    amp = logd * (1.0 / AMPLIFY)
    att = jnp.where(has, AMPLIFY / jnp.maximum(logd, 1e-5), zero)
    amp4 = jnp.concatenate([amp] * 4, axis=1)
    att4 = jnp.concatenate([att] * 4, axis=1)
    scaled = jnp.concatenate([agg, agg * amp4, agg * att4], axis=1)
    h = jnp.dot(scaled, u1w_ref[...], preferred_element_type=_F32) + u1b_ref[...]
    o_ref[...] = jnp.maximum(h, 0.0)


def _pad_rows(x, nt):
    return x if x.shape[0] == nt else jnp.pad(x, ((0, nt - x.shape[0]), (0, 0)))


def _node(degb, tab, u1w, u1b, *, block_rows=1024):
    n = tab.shape[0]
    block_rows = min(block_rows, -(-n // 8) * 8)
    nt = -(-n // block_rows) * block_rows
    degb = _pad_rows(degb, nt)
    tab = _pad_rows(tab, nt)
    spec = pl.BlockSpec((block_rows, H), lambda i: (i, 0))
    out = pl.pallas_call(
        _node_kernel,
        grid=(nt // block_rows,),
        in_specs=[spec,
                  pl.BlockSpec((block_rows, 4 * H), lambda i: (i, 0)),
                  pl.BlockSpec((12 * H, H), lambda i: (0, 0)),
                  pl.BlockSpec((1, H), lambda i: (0, 0))],
        out_specs=spec,
        out_shape=jax.ShapeDtypeStruct((nt, H), _F32),
    )(degb, tab, u1w, u1b.reshape(1, H))
    return out[:n] if nt != n else out


def _head_kernel(x_ref, w1_ref, b1_ref, w2_ref, b2_ref, w3_ref, b3_ref,
                 o_ref):
    h = jnp.dot(x_ref[...], w1_ref[...], preferred_element_type=_F32) + b1_ref[...]
    h = jnp.maximum(h, 0.0)
    h = jnp.dot(h, w2_ref[...], preferred_element_type=_F32) + b2_ref[...]
    h = jnp.maximum(h, 0.0)
    z = jnp.dot(h, w3_ref[...], preferred_element_type=_F32) + b3_ref[...]
    o_ref[...] = 1.0 / (1.0 + jnp.exp(-z))


def _head(x, w1, b1, w2, b2, w3, b3, *, block_rows=2000):
    n = x.shape[0]
    block_rows = min(block_rows, -(-n // 8) * 8)
    nt = -(-n // block_rows) * block_rows
    x = _pad_rows(x, nt)
    spec = pl.BlockSpec((block_rows, H), lambda i: (i, 0))
    wspec = pl.BlockSpec((H, H), lambda i: (0, 0))
    bspec = pl.BlockSpec((1, H), lambda i: (0, 0))
    out = pl.pallas_call(
        _head_kernel,
        grid=(nt // block_rows,),
        in_specs=[spec, wspec, bspec, wspec, bspec, wspec, bspec],
        out_specs=spec,
        out_shape=jax.ShapeDtypeStruct((nt, H), _F32),
    )(x, w1, b1.reshape(1, H), w2, b2.reshape(1, H), w3, b3.reshape(1, H))
    return out[:n] if nt != n else out


def _pna_hidden(At, Bn, tgt_g, nbr_g, seg, offs, degb, m2w, m2b, u1w, u1b):
    pre = _gather_pre(At, Bn, tgt_g, nbr_g)
    m = _mm(pre, m2w, m2b, relu_in=True, block_rows=2560)
    tab = _seg_reduce(m, seg, offs)
    return _node(degb, tab, u1w, u1b)


def kernel(demand, fac_init, adj, params):
    C = demand.shape[0]
    F = fac_init.shape[0]
    dst = adj[0]
    src = adj[1]

    E = dst.shape[0]
    perm_c = jnp.argsort(dst)
    dst_c = dst[perm_c]
    src_c = src[perm_c]
    perm_f = jnp.argsort(src)
    src_f = src[perm_f]
    dst_f = dst[perm_f]
    offs_c = jnp.searchsorted(dst_c, jnp.arange(C + 1, dtype=jnp.int32))
    deg_c = jnp.diff(offs_c).astype(_F32)
    offs_f = jnp.searchsorted(src_f, jnp.arange(F + 1, dtype=jnp.int32))
    deg_f = jnp.diff(offs_f).astype(_F32)
    degb_c = jnp.broadcast_to(deg_c[:, None], (C, H))
    degb_f = jnp.broadcast_to(deg_f[:, None], (F, H))

    ep = -(-(E + 160) // (_NW * _KE)) * (_NW * _KE)

    def _padi(x, v):
        return jnp.pad(x, (0, ep - E), constant_values=v) if ep != E else x

    dstc_g, srcc_g, seg_c = _padi(dst_c, 0), _padi(src_c, 0), _padi(dst_c, C)
    srcf_g, dstf_g, seg_f = _padi(src_f, 0), _padi(dst_f, 0), _padi(src_f, F)

    npad = 2 * _NW * _NPP
    osz = npad + 16

    def _pado(offs, n):
        return jnp.concatenate(
            [offs, jnp.full((osz - n - 1,), E, jnp.int32)]).astype(jnp.int32)

    offs_ec = _pado(offs_c, C)
    offs_ef = _pado(offs_f, F)
    degb_c = _pad_rows(degb_c, npad)
    degb_f = _pad_rows(degb_f, npad)

    p = params
    wce, bce = p["cus_emv"]["W"], p["cus_emv"]["b"]
    wfe, bfe = p["fac_emv"]["W"], p["fac_emv"]["b"]
    cp, fp = p["cus_pna"], p["fac_pna"]
    wt_c, wn_c, b1_c = cp["M1"]["W"][:H], cp["M1"]["W"][H:], cp["M1"]["b"]
    wt_f, wn_f, b1_f = fp["M1"]["W"][:H], fp["M1"]["W"][H:], fp["M1"]["b"]
    m2w_c, m2b_c = cp["M2"]["W"], cp["M2"]["b"]
    m2w_f, m2b_f = fp["M2"]["W"], fp["M2"]["b"]
    u1w_c, u1b_c = cp["U1"]["W"], cp["U1"]["b"]
    u1w_f, u1b_f = fp["U1"]["W"], fp["U1"]["b"]
    u2w_c, u2b_c = cp["U2"]["W"], cp["U2"]["b"]
    u2w_f, u2b_f = fp["U2"]["W"], fp["U2"]["b"]

    w_atc, b_atc = u2w_c @ wt_c, u2b_c @ wt_c + b1_c
    w_bnc, b_bnc = u2w_c @ wn_f, u2b_c @ wn_f
    w_atf, b_atf = u2w_f @ wt_f, u2b_f @ wt_f + b1_f
    w_bnf, b_bnf = u2w_f @ wn_c, u2b_f @ wn_c

    at_c = _mm(demand, wce @ wt_c, bce @ wt_c + b1_c)
    bn_c = _mm(demand, wce @ wn_f, bce @ wn_f)
    at_f = _mm(fac_init, wfe @ wt_f, bfe @ wt_f + b1_f)
    bn_f = _mm(fac_init, wfe @ wn_c, bfe @ wn_c)

    h_c = _pna_hidden(at_c, bn_f, dstc_g, srcc_g, seg_c, offs_ec, degb_c,
                      m2w_c, m2b_c, u1w_c, u1b_c)
    h_f = _pna_hidden(at_f, bn_c, srcf_g, dstf_g, seg_f, offs_ef, degb_f,
                      m2w_f, m2b_f, u1w_f, u1b_f)
    at_c = _mm(h_c, w_atc, b_atc)
    bn_c = _mm(h_c, w_bnc, b_bnc)
    at_f = _mm(h_f, w_atf, b_atf)
    bn_f = _mm(h_f, w_bnf, b_bnf)

    h_c = _pna_hidden(at_c, bn_f, dstc_g, srcc_g, seg_c, offs_ec, degb_c,
                      m2w_c, m2b_c, u1w_c, u1b_c)
    h_f = _pna_hidden(at_f, bn_c, srcf_g, dstf_g, seg_f, offs_ef, degb_f,
                      m2w_f, m2b_f, u1w_f, u1b_f)
    bn_c = _mm(h_c, w_bnc, b_bnc)
    at_f = _mm(h_f, w_atf, b_atf)

    h_f = _pna_hidden(at_f, bn_c, srcf_g, dstf_g, seg_f, offs_ef, degb_f,
                      m2w_f, m2b_f, u1w_f, u1b_f)
    f3 = _mm(h_f, u2w_f, u2b_f)

    f1w = jnp.zeros((H, H), _F32).at[:, :12].set(p["f1"]["W"])
    f1b = jnp.zeros((H,), _F32).at[:12].set(p["f1"]["b"])
    f2w = jnp.zeros((H, H), _F32).at[:12, :12].set(p["f2"]["W"])
    f2b = jnp.zeros((H,), _F32).at[:12].set(p["f2"]["b"])
    f3w = jnp.zeros((H, H), _F32).at[:12, :1].set(p["f3"]["W"])
    f3b = jnp.zeros((H,), _F32).at[:1].set(p["f3"]["b"])
    out = _head(f3, f1w, f1b, f2w, f2b, f3w, f3b)
    return out[:F, :1]


# speedup vs baseline: 2.6706x; 1.0004x over previous
import functools

import jax
import jax.numpy as jnp
from jax import lax
from jax.experimental import pallas as pl
from jax.experimental.pallas import tpu as pltpu
from jax.experimental.pallas import tpu_sc as plsc

H = 128
AMPLIFY = 3.5
_F32 = jnp.float32

_NC, _NS, _LANES = 2, 16, 16
_NW = _NC * _NS
_KE = 128


def _gather_pre(at_tab, bn_tab, tgt_idx, nbr_idx):
    e = tgt_idx.shape[0]
    g = e // _NW
    nch = g // _KE
    mesh = plsc.VectorSubcoreMesh(core_axis_name="c", subcore_axis_name="s")

    @functools.partial(
        pl.kernel,
        mesh=mesh,
        out_type=jax.ShapeDtypeStruct((e, H), _F32),
        scratch_types=[
            pltpu.VMEM((g,), jnp.int32),
            pltpu.VMEM((g,), jnp.int32),
            pltpu.VMEM((_KE, H), _F32),
            pltpu.VMEM((_KE, H), _F32),
            pltpu.VMEM((_KE, H), _F32),
            pltpu.VMEM((_KE, H), _F32),
            pltpu.VMEM((_KE, H), _F32),
            pltpu.VMEM((_KE, H), _F32),
            pltpu.SemaphoreType.DMA,
            pltpu.SemaphoreType.DMA,
            pltpu.SemaphoreType.DMA,
            pltpu.SemaphoreType.DMA,
            pltpu.SemaphoreType.DMA,
            pltpu.SemaphoreType.DMA,
        ],
    )
    def k(at_hbm, bn_hbm, ti_hbm, ni_hbm, out_hbm,
          ia, ib, ra0, ra1, rb0, rb1, ro0, ro1,
          sa0, sa1, sb0, sb1, so0, so1):
        wid = lax.axis_index("s") * _NC + lax.axis_index("c")
        base = wid * g
        ras, rbs, ros = (ra0, ra1), (rb0, rb1), (ro0, ro1)
        sas, sbs, sos = (sa0, sa1), (sb0, sb1), (so0, so1)
        pltpu.sync_copy(ti_hbm.at[pl.ds(base, g)], ia)
        pltpu.sync_copy(ni_hbm.at[pl.ds(base, g)], ib)

        def issue(ch, b):
            c = jnp.minimum(ch, nch - 1)
            sl = pl.ds(c * _KE, _KE)
            pltpu.async_copy(at_hbm.at[ia.at[sl]], ras[b], sas[b])
            pltpu.async_copy(bn_hbm.at[ib.at[sl]], rbs[b], sbs[b])

        def wait_g(b):
            pltpu.make_async_copy(at_hbm.at[pl.ds(0, _KE)], ras[b],
                                  sas[b]).wait()
            pltpu.make_async_copy(bn_hbm.at[pl.ds(0, _KE)], rbs[b],
                                  sbs[b]).wait()

        def compute(b):
            ra, rb, ro = ras[b], rbs[b], ros[b]

            def add_row(r, c2):
                for j in range(H // _LANES):
                    sl = pl.ds(j * _LANES, _LANES)
                    ro[r, sl] = ra[r, sl] + rb[r, sl]
                return c2

            lax.fori_loop(0, _KE, add_row, 0, unroll=2)

        def out_dma(ch, b):
            eoff = base + ch * _KE
            return pltpu.async_copy(ros[b], out_hbm.at[pl.ds(eoff, _KE)],
                                    sos[b])

        def wait_o(b):
            pltpu.make_async_copy(ros[b], out_hbm.at[pl.ds(0, _KE)],
                                  sos[b]).wait()

        if nch < 4:
            def sbody(ch, carry):
                issue(ch, 0)
                wait_g(0)
                compute(0)
                out_dma(ch, 0)
                wait_o(0)
                return carry

            lax.fori_loop(0, nch, sbody, 0)
            return

        issue(0, 0)
        issue(1, 1)
        wait_g(0)
        compute(0)
        out_dma(0, 0)
        issue(2, 0)
        wait_g(1)
        compute(1)
        out_dma(1, 1)
        issue(3, 1)

        def body(p, carry):
            for b in range(2):
                ch = 2 * p + b
                wait_g(b)
                wait_o(b)
                compute(b)
                out_dma(ch, b)
                issue(ch + 2, b)
            return carry

        lax.fori_loop(1, nch // 2, body, 0)
        if nch % 2:
            wait_g(0)
            wait_o(0)
            compute(0)
            out_dma(nch - 1, 0)
            wait_o(0)
            wait_o(1)
            wait_g(1)
        else:
            wait_o(0)
            wait_o(1)
            wait_g(0)
            wait_g(1)

    return k(at_tab, bn_tab, tgt_idx, nbr_idx)


_NPP = 160
_CH = 128


def _seg_reduce(m, seg, offs):
    npad = 2 * _NW * _NPP
    row_w = 4 * H
    mesh = plsc.VectorSubcoreMesh(core_axis_name="c", subcore_axis_name="s")

    @functools.partial(
        pl.kernel,
        mesh=mesh,
        out_type=jax.ShapeDtypeStruct((npad * row_w,), _F32),
        scratch_types=[
            pltpu.VMEM(((_NPP + 1) * row_w,), _F32),
            pltpu.VMEM((_CH * H,), _F32),
            pltpu.VMEM((_CH * H,), _F32),
            pltpu.VMEM((_CH + 16,), jnp.int32),
            pltpu.VMEM((_CH + 16,), jnp.int32),
            pltpu.VMEM((16,), jnp.int32),
            pltpu.VMEM((16,), jnp.int32),
            pltpu.SemaphoreType.DMA,
            pltpu.SemaphoreType.DMA,
            pltpu.SemaphoreType.DMA,
            pltpu.SemaphoreType.DMA,
        ],
    )
    def k(m_hbm, seg_hbm, offs_hbm, out_hbm, stg, mb0, mb1, sb0, sb1,
          ob0, ob1, sm0, sm1, sg0, sg1):
        mbufs, sbufs = (mb0, mb1), (sb0, sb1)
        sms, sgs = (sm0, sm1), (sg0, sg1)
        wid = lax.axis_index("s") * _NC + lax.axis_index("c")
        zero = jnp.zeros((_LANES,), _F32)
        ident = ((zero,) * 16
                 + (jnp.full((_LANES,), -3e38, _F32),) * 8
                 + (jnp.full((_LANES,), 3e38, _F32),) * 8)

        for p in range(2):
            v0 = pl.multiple_of((wid * 2 + p) * _NPP, 32)

            def zrow(r, c):
                for j in range(row_w // _LANES):
                    stg[pl.ds(r * row_w + j * _LANES, _LANES)] = zero
                return c

            lax.fori_loop(0, _NPP, zrow, 0)
            pltpu.sync_copy(offs_hbm.at[pl.ds(v0, 16)], ob0)
            pltpu.sync_copy(offs_hbm.at[pl.ds(v0 + _NPP, 16)], ob1)
            e0 = ob0[pl.ds(0, 16)][0]
            e1 = ob1[pl.ds(0, 16)][0]
            a0 = pl.multiple_of((e0 // 16) * 16, 16)
            nch = (e1 - a0 + (_CH - 1)) // _CH

            def issue_r(g, b):
                c = jnp.minimum(g, jnp.maximum(nch - 1, 0))
                cs = pl.multiple_of((a0 + c * _CH) // 16 * 16, 16)
                pltpu.async_copy(m_hbm.at[pl.ds(cs * H, _CH * H)],
                                 mbufs[b], sms[b])
                pltpu.async_copy(seg_hbm.at[pl.ds(cs, _CH + 16)],
                                 sbufs[b], sgs[b])

            def wait_r(b):
                pltpu.make_async_copy(m_hbm.at[pl.ds(0, _CH * H)],
                                      mbufs[b], sms[b]).wait()
                pltpu.make_async_copy(seg_hbm.at[pl.ds(0, _CH + 16)],
                                      sbufs[b], sgs[b]).wait()

            def process(g, b, carry):
                mbuf, sbuf = mbufs[b], sbufs[b]
                cs = a0 + g * _CH
                r_lo = jnp.maximum(0, e0 - cs)
                r_hi = jnp.minimum(_CH, e1 - cs)

                def edge_body(r, ec):
                    cur = ec[0]
                    sv = sbuf[pl.ds(r, 16)][0]
                    pred = jnp.logical_and(sv != cur, cur >= 0)
                    base = jnp.where(pred, cur - v0, _NPP) * row_w
                    for j in range(8):
                        for t in range(4):
                            stg[pl.ds(base + t * H + j * _LANES, _LANES)] = \
                                ec[1 + 8 * t + j]
                    new = [sv]
                    for j in range(8):
                        v = mbuf[pl.ds(r * H + j * _LANES, _LANES)]
                        s_ = jnp.where(pred, ident[j], ec[1 + j])
                        new.append(s_ + v)
                    for j in range(8):
                        v = mbuf[pl.ds(r * H + j * _LANES, _LANES)]
                        q_ = jnp.where(pred, ident[8 + j], ec[9 + j])
                        new.append(q_ + v * v)
                    for j in range(8):
                        v = mbuf[pl.ds(r * H + j * _LANES, _LANES)]
                        x_ = jnp.where(pred, ident[16 + j], ec[17 + j])
                        new.append(jnp.maximum(x_, v))
                    for j in range(8):
                        v = mbuf[pl.ds(r * H + j * _LANES, _LANES)]
                        n_ = jnp.where(pred, ident[24 + j], ec[25 + j])
                        new.append(jnp.minimum(n_, v))
                    return tuple(new)

                return lax.fori_loop(r_lo, r_hi, edge_body, carry)

            issue_r(0, 0)

            def pair_body(p, carry):
                for b in range(2):
                    g = 2 * p + b
                    issue_r(g + 1, 1 - b)
                    wait_r(b)
                    carry = process(g, b, carry)
                return carry

            fin = lax.fori_loop(0, (nch + 1) // 2, pair_body,
                                (jnp.int32(-1),) + ident)
            wait_r(0)
            fbase = jnp.where(fin[0] >= 0, fin[0] - v0, _NPP) * row_w
            for j in range(8):
                for t in range(4):
                    stg[pl.ds(fbase + t * H + j * _LANES, _LANES)] = \
                        fin[1 + 8 * t + j]
            pltpu.sync_copy(stg.at[pl.ds(0, _NPP * row_w)],
                            out_hbm.at[pl.ds(v0 * row_w, _NPP * row_w)])

    return k(m.reshape(-1), seg, offs).reshape(npad, row_w)


def _mm_kernel(x_ref, w_ref, b_ref, o_ref, *, relu_in, relu_out):
    x = x_ref[...]
    if relu_in:
        x = jnp.maximum(x, 0.0)
    y = jnp.dot(x, w_ref[...], preferred_element_type=_F32) + b_ref[...]
    if relu_out:
        y = jnp.maximum(y, 0.0)
    o_ref[...] = y


def _mm(x, w, b, *, relu_in=False, relu_out=False, block_rows=2000):
    n, k = x.shape
    m = w.shape[1]
    npad = -n % block_rows
    if npad:
        x = jnp.pad(x, ((0, npad), (0, 0)))
    nt = n + npad
    kern = functools.partial(_mm_kernel, relu_in=relu_in, relu_out=relu_out)
    out = pl.pallas_call(
        kern,
        grid=(nt // block_rows,),
        in_specs=[
            pl.BlockSpec((block_rows, k), lambda i: (i, 0)),
            pl.BlockSpec((k, m), lambda i: (0, 0)),
            pl.BlockSpec((1, m), lambda i: (0, 0)),
        ],
        out_specs=pl.BlockSpec((block_rows, m), lambda i: (i, 0)),
        out_shape=jax.ShapeDtypeStruct((nt, m), _F32),
    )(x, w, b.reshape(1, m))
    return out[:n] if npad else out


def _node_kernel(deg_ref, tab_ref, u1w_ref, u1b_ref, o_ref):
    deg = deg_ref[...]
    tab = tab_ref[...]
    s = tab[:, :H]
    ss = tab[:, H:2 * H]
    degc = jnp.maximum(deg, 1.0)
    mean = s / degc
    var = jnp.maximum(ss / degc - mean * mean, 0.0)
    std = jnp.sqrt(var + 1e-5)
    has = deg > 0.0
    zero = jnp.zeros_like(mean)
    mx = jnp.where(has, tab[:, 2 * H:3 * H], zero)
    mn = jnp.where(has, tab[:, 3 * H:], zero)
    agg = jnp.concatenate([mx, mn, mean, std], axis=1)
    logd = jnp.log(deg + 1.0)
    amp = logd * (1.0 / AMPLIFY)
    att = jnp.where(has, AMPLIFY / jnp.maximum(logd, 1e-5), zero)
    amp4 = jnp.concatenate([amp] * 4, axis=1)
    att4 = jnp.concatenate([att] * 4, axis=1)
    scaled = jnp.concatenate([agg, agg * amp4, agg * att4], axis=1)
    h = jnp.dot(scaled, u1w_ref[...], preferred_element_type=_F32) + u1b_ref[...]
    o_ref[...] = jnp.maximum(h, 0.0)


def _pad_rows(x, nt):
    return x if x.shape[0] == nt else jnp.pad(x, ((0, nt - x.shape[0]), (0, 0)))


def _node(degb, tab, u1w, u1b, *, block_rows=1024):
    n = tab.shape[0]
    block_rows = min(block_rows, -(-n // 8) * 8)
    nt = -(-n // block_rows) * block_rows
    degb = _pad_rows(degb, nt)
    tab = _pad_rows(tab, nt)
    spec = pl.BlockSpec((block_rows, H), lambda i: (i, 0))
    out = pl.pallas_call(
        _node_kernel,
        grid=(nt // block_rows,),
        in_specs=[spec,
                  pl.BlockSpec((block_rows, 4 * H), lambda i: (i, 0)),
                  pl.BlockSpec((12 * H, H), lambda i: (0, 0)),
                  pl.BlockSpec((1, H), lambda i: (0, 0))],
        out_specs=spec,
        out_shape=jax.ShapeDtypeStruct((nt, H), _F32),
    )(degb, tab, u1w, u1b.reshape(1, H))
    return out[:n] if nt != n else out


def _head_kernel(x_ref, w1_ref, b1_ref, w2_ref, b2_ref, w3_ref, b3_ref,
                 o_ref):
    h = jnp.dot(x_ref[...], w1_ref[...], preferred_element_type=_F32) + b1_ref[...]
    h = jnp.maximum(h, 0.0)
    h = jnp.dot(h, w2_ref[...], preferred_element_type=_F32) + b2_ref[...]
    h = jnp.maximum(h, 0.0)
    z = jnp.dot(h, w3_ref[...], preferred_element_type=_F32) + b3_ref[...]
    o_ref[...] = 1.0 / (1.0 + jnp.exp(-z))


def _head(x, w1, b1, w2, b2, w3, b3, *, block_rows=2000):
    n = x.shape[0]
    block_rows = min(block_rows, -(-n // 8) * 8)
    nt = -(-n // block_rows) * block_rows
    x = _pad_rows(x, nt)
    spec = pl.BlockSpec((block_rows, H), lambda i: (i, 0))
    wspec = pl.BlockSpec((H, H), lambda i: (0, 0))
    bspec = pl.BlockSpec((1, H), lambda i: (0, 0))
    out = pl.pallas_call(
        _head_kernel,
        grid=(nt // block_rows,),
        in_specs=[spec, wspec, bspec, wspec, bspec, wspec, bspec],
        out_specs=spec,
        out_shape=jax.ShapeDtypeStruct((nt, H), _F32),
    )(x, w1, b1.reshape(1, H), w2, b2.reshape(1, H), w3, b3.reshape(1, H))
    return out[:n] if nt != n else out


def _pna_hidden(At, Bn, tgt_g, nbr_g, seg, offs, degb, m2w, m2b, u1w, u1b):
    pre = _gather_pre(At, Bn, tgt_g, nbr_g)
    m = _mm(pre, m2w, m2b, relu_in=True, block_rows=2560)
    tab = _seg_reduce(m, seg, offs)
    return _node(degb, tab, u1w, u1b)


def kernel(demand, fac_init, adj, params):
    C = demand.shape[0]
    F = fac_init.shape[0]
    dst = adj[0]
    src = adj[1]

    E = dst.shape[0]
    perm_c = jnp.argsort(dst)
    dst_c = dst[perm_c]
    src_c = src[perm_c]
    perm_f = jnp.argsort(src)
    src_f = src[perm_f]
    dst_f = dst[perm_f]
    offs_c = jnp.searchsorted(dst_c, jnp.arange(C + 1, dtype=jnp.int32))
    deg_c = jnp.diff(offs_c).astype(_F32)
    offs_f = jnp.searchsorted(src_f, jnp.arange(F + 1, dtype=jnp.int32))
    deg_f = jnp.diff(offs_f).astype(_F32)
    degb_c = jnp.broadcast_to(deg_c[:, None], (C, H))
    degb_f = jnp.broadcast_to(deg_f[:, None], (F, H))

    ep = -(-(E + 160) // (_NW * _KE)) * (_NW * _KE)

    def _padi(x, v):
        return jnp.pad(x, (0, ep - E), constant_values=v) if ep != E else x

    dstc_g, srcc_g, seg_c = _padi(dst_c, 0), _padi(src_c, 0), _padi(dst_c, C)
    srcf_g, dstf_g, seg_f = _padi(src_f, 0), _padi(dst_f, 0), _padi(src_f, F)

    npad = 2 * _NW * _NPP
    osz = npad + 16

    def _pado(offs, n):
        return jnp.concatenate(
            [offs, jnp.full((osz - n - 1,), E, jnp.int32)]).astype(jnp.int32)

    offs_ec = _pado(offs_c, C)
    offs_ef = _pado(offs_f, F)
    degb_c = _pad_rows(degb_c, npad)
    degb_f = _pad_rows(degb_f, npad)

    p = params
    wce, bce = p["cus_emv"]["W"], p["cus_emv"]["b"]
    wfe, bfe = p["fac_emv"]["W"], p["fac_emv"]["b"]
    cp, fp = p["cus_pna"], p["fac_pna"]
    wt_c, wn_c, b1_c = cp["M1"]["W"][:H], cp["M1"]["W"][H:], cp["M1"]["b"]
    wt_f, wn_f, b1_f = fp["M1"]["W"][:H], fp["M1"]["W"][H:], fp["M1"]["b"]
    m2w_c, m2b_c = cp["M2"]["W"], cp["M2"]["b"]
    m2w_f, m2b_f = fp["M2"]["W"], fp["M2"]["b"]
    u1w_c, u1b_c = cp["U1"]["W"], cp["U1"]["b"]
    u1w_f, u1b_f = fp["U1"]["W"], fp["U1"]["b"]
    u2w_c, u2b_c = cp["U2"]["W"], cp["U2"]["b"]
    u2w_f, u2b_f = fp["U2"]["W"], fp["U2"]["b"]

    w_atc, b_atc = u2w_c @ wt_c, u2b_c @ wt_c + b1_c
    w_bnc, b_bnc = u2w_c @ wn_f, u2b_c @ wn_f
    w_atf, b_atf = u2w_f @ wt_f, u2b_f @ wt_f + b1_f
    w_bnf, b_bnf = u2w_f @ wn_c, u2b_f @ wn_c

    at_c = _mm(demand, wce @ wt_c, bce @ wt_c + b1_c)
    bn_c = _mm(demand, wce @ wn_f, bce @ wn_f)
    at_f = _mm(fac_init, wfe @ wt_f, bfe @ wt_f + b1_f)
    bn_f = _mm(fac_init, wfe @ wn_c, bfe @ wn_c)

    h_c = _pna_hidden(at_c, bn_f, dstc_g, srcc_g, seg_c, offs_ec, degb_c,
                      m2w_c, m2b_c, u1w_c, u1b_c)
    h_f = _pna_hidden(at_f, bn_c, srcf_g, dstf_g, seg_f, offs_ef, degb_f,
                      m2w_f, m2b_f, u1w_f, u1b_f)
    at_c = _mm(h_c, w_atc, b_atc)
    bn_c = _mm(h_c, w_bnc, b_bnc)
    at_f = _mm(h_f, w_atf, b_atf)
    bn_f = _mm(h_f, w_bnf, b_bnf)

    h_c = _pna_hidden(at_c, bn_f, dstc_g, srcc_g, seg_c, offs_ec, degb_c,
                      m2w_c, m2b_c, u1w_c, u1b_c)
    h_f = _pna_hidden(at_f, bn_c, srcf_g, dstf_g, seg_f, offs_ef, degb_f,
                      m2w_f, m2b_f, u1w_f, u1b_f)
    bn_c = _mm(h_c, w_bnc, b_bnc)
    at_f = _mm(h_f, w_atf, b_atf)

    h_f = _pna_hidden(at_f, bn_c, srcf_g, dstf_g, seg_f, offs_ef, degb_f,
                      m2w_f, m2b_f, u1w_f, u1b_f)
    f3 = _mm(h_f, u2w_f, u2b_f)

    f1w = jnp.zeros((H, H), _F32).at[:, :12].set(p["f1"]["W"])
    f1b = jnp.zeros((H,), _F32).at[:12].set(p["f1"]["b"])
    f2w = jnp.zeros((H, H), _F32).at[:12, :12].set(p["f2"]["W"])
    f2b = jnp.zeros((H,), _F32).at[:12].set(p["f2"]["b"])
    f3w = jnp.zeros((H, H), _F32).at[:12, :1].set(p["f3"]["W"])
    f3b = jnp.zeros((H,), _F32).at[:1].set(p["f3"]["b"])
    out = _head(f3, f1w, f1b, f2w, f2b, f3w, f3b)
    return out[:F, :1]


# speedup vs baseline: 3.0754x; 1.1516x over previous
import functools

import jax
import jax.numpy as jnp
from jax import lax
from jax.experimental import pallas as pl
from jax.experimental.pallas import tpu as pltpu
from jax.experimental.pallas import tpu_sc as plsc

H = 128
AMPLIFY = 3.5
_F32 = jnp.float32

_NC, _NS, _LANES = 2, 16, 16
_NW = _NC * _NS
_KE = 128


def _gather_pre(at_tab, bn_tab, tgt_idx, nbr_idx):
    e = tgt_idx.shape[0]
    g = e // _NW
    nch = g // _KE
    mesh = plsc.VectorSubcoreMesh(core_axis_name="c", subcore_axis_name="s")

    @functools.partial(
        pl.kernel,
        mesh=mesh,
        out_type=jax.ShapeDtypeStruct((e, H), _F32),
        scratch_types=[
            pltpu.VMEM((g,), jnp.int32),
            pltpu.VMEM((g,), jnp.int32),
            pltpu.VMEM((_KE, H), _F32),
            pltpu.VMEM((_KE, H), _F32),
            pltpu.VMEM((_KE, H), _F32),
            pltpu.VMEM((_KE, H), _F32),
            pltpu.VMEM((_KE, H), _F32),
            pltpu.VMEM((_KE, H), _F32),
            pltpu.SemaphoreType.DMA,
            pltpu.SemaphoreType.DMA,
            pltpu.SemaphoreType.DMA,
            pltpu.SemaphoreType.DMA,
            pltpu.SemaphoreType.DMA,
            pltpu.SemaphoreType.DMA,
        ],
    )
    def k(at_hbm, bn_hbm, ti_hbm, ni_hbm, out_hbm,
          ia, ib, ra0, ra1, rb0, rb1, ro0, ro1,
          sa0, sa1, sb0, sb1, so0, so1):
        wid = lax.axis_index("s") * _NC + lax.axis_index("c")
        base = wid * g
        ras, rbs, ros = (ra0, ra1), (rb0, rb1), (ro0, ro1)
        sas, sbs, sos = (sa0, sa1), (sb0, sb1), (so0, so1)
        pltpu.sync_copy(ti_hbm.at[pl.ds(base, g)], ia)
        pltpu.sync_copy(ni_hbm.at[pl.ds(base, g)], ib)

        def issue(ch, b):
            c = jnp.minimum(ch, nch - 1)
            sl = pl.ds(c * _KE, _KE)
            pltpu.async_copy(at_hbm.at[ia.at[sl]], ras[b], sas[b])
            pltpu.async_copy(bn_hbm.at[ib.at[sl]], rbs[b], sbs[b])

        def wait_g(b):
            pltpu.make_async_copy(at_hbm.at[pl.ds(0, _KE)], ras[b],
                                  sas[b]).wait()
            pltpu.make_async_copy(bn_hbm.at[pl.ds(0, _KE)], rbs[b],
                                  sbs[b]).wait()

        def compute(b):
            ra, rb, ro = ras[b], rbs[b], ros[b]

            def add_row(r, c2):
                for j in range(H // _LANES):
                    sl = pl.ds(j * _LANES, _LANES)
                    ro[r, sl] = ra[r, sl] + rb[r, sl]
                return c2

            lax.fori_loop(0, _KE, add_row, 0, unroll=2)

        def out_dma(ch, b):
            eoff = base + ch * _KE
            return pltpu.async_copy(ros[b], out_hbm.at[pl.ds(eoff, _KE)],
                                    sos[b])

        def wait_o(b):
            pltpu.make_async_copy(ros[b], out_hbm.at[pl.ds(0, _KE)],
                                  sos[b]).wait()

        if nch < 4:
            def sbody(ch, carry):
                issue(ch, 0)
                wait_g(0)
                compute(0)
                out_dma(ch, 0)
                wait_o(0)
                return carry

            lax.fori_loop(0, nch, sbody, 0)
            return

        issue(0, 0)
        issue(1, 1)
        wait_g(0)
        compute(0)
        out_dma(0, 0)
        issue(2, 0)
        wait_g(1)
        compute(1)
        out_dma(1, 1)
        issue(3, 1)

        def body(p, carry):
            for b in range(2):
                ch = 2 * p + b
                wait_g(b)
                wait_o(b)
                compute(b)
                out_dma(ch, b)
                issue(ch + 2, b)
            return carry

        lax.fori_loop(1, nch // 2, body, 0)
        if nch % 2:
            wait_g(0)
            wait_o(0)
            compute(0)
            out_dma(nch - 1, 0)
            wait_o(0)
            wait_o(1)
            wait_g(1)
        else:
            wait_o(0)
            wait_o(1)
            wait_g(0)
            wait_g(1)

    return k(at_tab, bn_tab, tgt_idx, nbr_idx)


_NPP = 80
_NPH = 4
_CH = 256


def _seg_reduce(m, seg, offs):
    npad = _NPH * _NW * _NPP
    row_w = 4 * H
    mesh = plsc.VectorSubcoreMesh(core_axis_name="c", subcore_axis_name="s")

    @functools.partial(
        pl.kernel,
        mesh=mesh,
        out_type=jax.ShapeDtypeStruct((npad * row_w,), _F32),
        scratch_types=[
            pltpu.VMEM((_NPP * row_w,), _F32),
            pltpu.VMEM((_CH * H,), _F32),
            pltpu.VMEM((_CH * H,), _F32),
            pltpu.VMEM((_CH + 16,), jnp.int32),
            pltpu.VMEM((_CH + 16,), jnp.int32),
            pltpu.VMEM((_NPP + 16,), jnp.int32),
            pltpu.SemaphoreType.DMA,
            pltpu.SemaphoreType.DMA,
            pltpu.SemaphoreType.DMA,
            pltpu.SemaphoreType.DMA,
        ],
    )
    def k(m_hbm, seg_hbm, offs_hbm, out_hbm, stg, mb0, mb1, sb0, sb1,
          obuf, sm0, sm1, sg0, sg1):
        mbufs, sbufs = (mb0, mb1), (sb0, sb1)
        sms, sgs = (sm0, sm1), (sg0, sg1)
        wid = lax.axis_index("s") * _NC + lax.axis_index("c")
        zero = jnp.zeros((_LANES,), _F32)
        ident = ((zero,) * 16
                 + (jnp.full((_LANES,), -3e38, _F32),) * 8
                 + (jnp.full((_LANES,), 3e38, _F32),) * 8)

        for p in range(_NPH):
            v0 = pl.multiple_of((wid * _NPH + p) * _NPP, 16)

            def zrow(r, c):
                for j in range(row_w // _LANES):
                    stg[pl.ds(r * row_w + j * _LANES, _LANES)] = zero
                return c

            lax.fori_loop(0, _NPP, zrow, 0)
            pltpu.sync_copy(offs_hbm.at[pl.ds(v0, _NPP + 16)], obuf)
            e0 = obuf[pl.ds(0, 16)][0]
            e1 = obuf[pl.ds(_NPP, 16)][0]
            a0 = pl.multiple_of((e0 // 16) * 16, 16)
            nch = (e1 - a0 + (_CH - 1)) // _CH

            def issue_r(g, b):
                c = jnp.minimum(g, jnp.maximum(nch - 1, 0))
                cs = pl.multiple_of((a0 + c * _CH) // 16 * 16, 16)
                pltpu.async_copy(m_hbm.at[pl.ds(cs * H, _CH * H)],
                                 mbufs[b], sms[b])
                pltpu.async_copy(seg_hbm.at[pl.ds(cs, _CH + 16)],
                                 sbufs[b], sgs[b])

            def wait_r(b):
                pltpu.make_async_copy(m_hbm.at[pl.ds(0, _CH * H)],
                                      mbufs[b], sms[b]).wait()
                pltpu.make_async_copy(seg_hbm.at[pl.ds(0, _CH + 16)],
                                      sbufs[b], sgs[b]).wait()

            def process(g, b, carry):
                mbuf, sbuf = mbufs[b], sbufs[b]
                cs = a0 + g * _CH
                r_lo = jnp.maximum(0, e0 - cs)
                r_hi = jnp.minimum(_CH, e1 - cs)
                nonempty = r_hi > r_lo
                vlo = sbuf[pl.ds(r_lo, 16)][0]
                vhi = sbuf[pl.ds(jnp.maximum(r_hi - 1, 0), 16)][0]
                vend = jnp.where(nonempty, vhi + 1, vlo)

                def node_body(v, nc):
                    o0 = obuf[pl.ds(v - v0, 16)][0]
                    o1 = obuf[pl.ds(v - v0 + 1, 16)][0]
                    lo = jnp.maximum(o0 - cs, r_lo)
                    hi = jnp.minimum(o1 - cs, r_hi)
                    fresh = o0 - cs >= r_lo
                    acc = tuple(
                        jnp.where(fresh, ident[j], nc[j]) for j in range(32))

                    def eb(r, ac):
                        out = []
                        for j in range(8):
                            v_ = mbuf[pl.ds(r * H + j * _LANES, _LANES)]
                            out.append(ac[j] + v_)
                        for j in range(8):
                            v_ = mbuf[pl.ds(r * H + j * _LANES, _LANES)]
                            out.append(ac[8 + j] + v_ * v_)
                        for j in range(8):
                            v_ = mbuf[pl.ds(r * H + j * _LANES, _LANES)]
                            out.append(jnp.maximum(ac[16 + j], v_))
                        for j in range(8):
                            v_ = mbuf[pl.ds(r * H + j * _LANES, _LANES)]
                            out.append(jnp.minimum(ac[24 + j], v_))
                        return tuple(out)

                    acc = lax.fori_loop(lo, hi, eb, acc)
                    base = (v - v0) * row_w
                    for t in range(4):
                        for j in range(8):
                            stg[pl.ds(base + t * H + j * _LANES, _LANES)] = \
                                acc[8 * t + j]
                    return acc

                return lax.fori_loop(vlo, vend, node_body, carry)

            issue_r(0, 0)

            def pair_body(pp, carry):
                for b in range(2):
                    g = 2 * pp + b
                    issue_r(g + 1, 1 - b)
                    wait_r(b)
                    carry = process(g, b, carry)
                return carry

            lax.fori_loop(0, (nch + 1) // 2, pair_body, ident)
            wait_r(0)
            pltpu.sync_copy(stg, out_hbm.at[pl.ds(v0 * row_w, _NPP * row_w)])

    return k(m.reshape(-1), seg, offs).reshape(npad, row_w)


def _mm_kernel(x_ref, w_ref, b_ref, o_ref, *, relu_in, relu_out):
    x = x_ref[...]
    if relu_in:
        x = jnp.maximum(x, 0.0)
    y = jnp.dot(x, w_ref[...], preferred_element_type=_F32) + b_ref[...]
    if relu_out:
        y = jnp.maximum(y, 0.0)
    o_ref[...] = y


def _mm(x, w, b, *, relu_in=False, relu_out=False, block_rows=2000):
    n, k = x.shape
    m = w.shape[1]
    npad = -n % block_rows
    if npad:
        x = jnp.pad(x, ((0, npad), (0, 0)))
    nt = n + npad
    kern = functools.partial(_mm_kernel, relu_in=relu_in, relu_out=relu_out)
    out = pl.pallas_call(
        kern,
        grid=(nt // block_rows,),
        in_specs=[
            pl.BlockSpec((block_rows, k), lambda i: (i, 0)),
            pl.BlockSpec((k, m), lambda i: (0, 0)),
            pl.BlockSpec((1, m), lambda i: (0, 0)),
        ],
        out_specs=pl.BlockSpec((block_rows, m), lambda i: (i, 0)),
        out_shape=jax.ShapeDtypeStruct((nt, m), _F32),
    )(x, w, b.reshape(1, m))
    return out[:n] if npad else out


def _node_kernel(deg_ref, tab_ref, u1w_ref, u1b_ref, o_ref):
    deg = deg_ref[...]
    tab = tab_ref[...]
    s = tab[:, :H]
    ss = tab[:, H:2 * H]
    degc = jnp.maximum(deg, 1.0)
    mean = s / degc
    var = jnp.maximum(ss / degc - mean * mean, 0.0)
    std = jnp.sqrt(var + 1e-5)
    has = deg > 0.0
    zero = jnp.zeros_like(mean)
    mx = jnp.where(has, tab[:, 2 * H:3 * H], zero)
    mn = jnp.where(has, tab[:, 3 * H:], zero)
    agg = jnp.concatenate([mx, mn, mean, std], axis=1)
    logd = jnp.log(deg + 1.0)
    amp = logd * (1.0 / AMPLIFY)
    att = jnp.where(has, AMPLIFY / jnp.maximum(logd, 1e-5), zero)
    amp4 = jnp.concatenate([amp] * 4, axis=1)
    att4 = jnp.concatenate([att] * 4, axis=1)
    scaled = jnp.concatenate([agg, agg * amp4, agg * att4], axis=1)
    h = jnp.dot(scaled, u1w_ref[...], preferred_element_type=_F32) + u1b_ref[...]
    o_ref[...] = jnp.maximum(h, 0.0)


def _pad_rows(x, nt):
    return x if x.shape[0] == nt else jnp.pad(x, ((0, nt - x.shape[0]), (0, 0)))


def _node(degb, tab, u1w, u1b, *, block_rows=1024):
    n = tab.shape[0]
    block_rows = min(block_rows, -(-n // 8) * 8)
    nt = -(-n // block_rows) * block_rows
    degb = _pad_rows(degb, nt)
    tab = _pad_rows(tab, nt)
    spec = pl.BlockSpec((block_rows, H), lambda i: (i, 0))
    out = pl.pallas_call(
        _node_kernel,
        grid=(nt // block_rows,),
        in_specs=[spec,
                  pl.BlockSpec((block_rows, 4 * H), lambda i: (i, 0)),
                  pl.BlockSpec((12 * H, H), lambda i: (0, 0)),
                  pl.BlockSpec((1, H), lambda i: (0, 0))],
        out_specs=spec,
        out_shape=jax.ShapeDtypeStruct((nt, H), _F32),
    )(degb, tab, u1w, u1b.reshape(1, H))
    return out[:n] if nt != n else out


def _head_kernel(x_ref, w1_ref, b1_ref, w2_ref, b2_ref, w3_ref, b3_ref,
                 o_ref):
    h = jnp.dot(x_ref[...], w1_ref[...], preferred_element_type=_F32) + b1_ref[...]
    h = jnp.maximum(h, 0.0)
    h = jnp.dot(h, w2_ref[...], preferred_element_type=_F32) + b2_ref[...]
    h = jnp.maximum(h, 0.0)
    z = jnp.dot(h, w3_ref[...], preferred_element_type=_F32) + b3_ref[...]
    o_ref[...] = 1.0 / (1.0 + jnp.exp(-z))


def _head(x, w1, b1, w2, b2, w3, b3, *, block_rows=2000):
    n = x.shape[0]
    block_rows = min(block_rows, -(-n // 8) * 8)
    nt = -(-n // block_rows) * block_rows
    x = _pad_rows(x, nt)
    spec = pl.BlockSpec((block_rows, H), lambda i: (i, 0))
    wspec = pl.BlockSpec((H, H), lambda i: (0, 0))
    bspec = pl.BlockSpec((1, H), lambda i: (0, 0))
    out = pl.pallas_call(
        _head_kernel,
        grid=(nt // block_rows,),
        in_specs=[spec, wspec, bspec, wspec, bspec, wspec, bspec],
        out_specs=spec,
        out_shape=jax.ShapeDtypeStruct((nt, H), _F32),
    )(x, w1, b1.reshape(1, H), w2, b2.reshape(1, H), w3, b3.reshape(1, H))
    return out[:n] if nt != n else out


def _pna_hidden(At, Bn, tgt_g, nbr_g, seg, offs, degb, m2w, m2b, u1w, u1b):
    pre = _gather_pre(At, Bn, tgt_g, nbr_g)
    m = _mm(pre, m2w, m2b, relu_in=True, block_rows=2560)
    tab = _seg_reduce(m, seg, offs)
    return _node(degb, tab, u1w, u1b)


def kernel(demand, fac_init, adj, params):
    C = demand.shape[0]
    F = fac_init.shape[0]
    dst = adj[0]
    src = adj[1]

    E = dst.shape[0]
    perm_c = jnp.argsort(dst)
    dst_c = dst[perm_c]
    src_c = src[perm_c]
    perm_f = jnp.argsort(src)
    src_f = src[perm_f]
    dst_f = dst[perm_f]
    offs_c = jnp.searchsorted(dst_c, jnp.arange(C + 1, dtype=jnp.int32))
    deg_c = jnp.diff(offs_c).astype(_F32)
    offs_f = jnp.searchsorted(src_f, jnp.arange(F + 1, dtype=jnp.int32))
    deg_f = jnp.diff(offs_f).astype(_F32)
    degb_c = jnp.broadcast_to(deg_c[:, None], (C, H))
    degb_f = jnp.broadcast_to(deg_f[:, None], (F, H))

    ep = -(-(E + 160) // (_NW * _KE)) * (_NW * _KE)

    def _padi(x, v):
        return jnp.pad(x, (0, ep - E), constant_values=v) if ep != E else x

    dstc_g, srcc_g, seg_c = _padi(dst_c, 0), _padi(src_c, 0), _padi(dst_c, C)
    srcf_g, dstf_g, seg_f = _padi(src_f, 0), _padi(dst_f, 0), _padi(src_f, F)

    npad = _NPH * _NW * _NPP
    osz = npad + 16

    def _pado(offs, n):
        return jnp.concatenate(
            [offs, jnp.full((osz - n - 1,), E, jnp.int32)]).astype(jnp.int32)

    offs_ec = _pado(offs_c, C)
    offs_ef = _pado(offs_f, F)
    degb_c = _pad_rows(degb_c, npad)
    degb_f = _pad_rows(degb_f, npad)

    p = params
    wce, bce = p["cus_emv"]["W"], p["cus_emv"]["b"]
    wfe, bfe = p["fac_emv"]["W"], p["fac_emv"]["b"]
    cp, fp = p["cus_pna"], p["fac_pna"]
    wt_c, wn_c, b1_c = cp["M1"]["W"][:H], cp["M1"]["W"][H:], cp["M1"]["b"]
    wt_f, wn_f, b1_f = fp["M1"]["W"][:H], fp["M1"]["W"][H:], fp["M1"]["b"]
    m2w_c, m2b_c = cp["M2"]["W"], cp["M2"]["b"]
    m2w_f, m2b_f = fp["M2"]["W"], fp["M2"]["b"]
    u1w_c, u1b_c = cp["U1"]["W"], cp["U1"]["b"]
    u1w_f, u1b_f = fp["U1"]["W"], fp["U1"]["b"]
    u2w_c, u2b_c = cp["U2"]["W"], cp["U2"]["b"]
    u2w_f, u2b_f = fp["U2"]["W"], fp["U2"]["b"]

    w_atc, b_atc = u2w_c @ wt_c, u2b_c @ wt_c + b1_c
    w_bnc, b_bnc = u2w_c @ wn_f, u2b_c @ wn_f
    w_atf, b_atf = u2w_f @ wt_f, u2b_f @ wt_f + b1_f
    w_bnf, b_bnf = u2w_f @ wn_c, u2b_f @ wn_c

    at_c = _mm(demand, wce @ wt_c, bce @ wt_c + b1_c)
    bn_c = _mm(demand, wce @ wn_f, bce @ wn_f)
    at_f = _mm(fac_init, wfe @ wt_f, bfe @ wt_f + b1_f)
    bn_f = _mm(fac_init, wfe @ wn_c, bfe @ wn_c)

    h_c = _pna_hidden(at_c, bn_f, dstc_g, srcc_g, seg_c, offs_ec, degb_c,
                      m2w_c, m2b_c, u1w_c, u1b_c)
    h_f = _pna_hidden(at_f, bn_c, srcf_g, dstf_g, seg_f, offs_ef, degb_f,
                      m2w_f, m2b_f, u1w_f, u1b_f)
    at_c = _mm(h_c, w_atc, b_atc)
    bn_c = _mm(h_c, w_bnc, b_bnc)
    at_f = _mm(h_f, w_atf, b_atf)
    bn_f = _mm(h_f, w_bnf, b_bnf)

    h_c = _pna_hidden(at_c, bn_f, dstc_g, srcc_g, seg_c, offs_ec, degb_c,
                      m2w_c, m2b_c, u1w_c, u1b_c)
    h_f = _pna_hidden(at_f, bn_c, srcf_g, dstf_g, seg_f, offs_ef, degb_f,
                      m2w_f, m2b_f, u1w_f, u1b_f)
    bn_c = _mm(h_c, w_bnc, b_bnc)
    at_f = _mm(h_f, w_atf, b_atf)

    h_f = _pna_hidden(at_f, bn_c, srcf_g, dstf_g, seg_f, offs_ef, degb_f,
                      m2w_f, m2b_f, u1w_f, u1b_f)
    f3 = _mm(h_f, u2w_f, u2b_f)

    f1w = jnp.zeros((H, H), _F32).at[:, :12].set(p["f1"]["W"])
    f1b = jnp.zeros((H,), _F32).at[:12].set(p["f1"]["b"])
    f2w = jnp.zeros((H, H), _F32).at[:12, :12].set(p["f2"]["W"])
    f2b = jnp.zeros((H,), _F32).at[:12].set(p["f2"]["b"])
    f3w = jnp.zeros((H, H), _F32).at[:12, :1].set(p["f3"]["W"])
    f3b = jnp.zeros((H,), _F32).at[:1].set(p["f3"]["b"])
    out = _head(f3, f1w, f1b, f2w, f2b, f3w, f3b)
    return out[:F, :1]
